# baseline (speedup 1.0000x reference)
import jax
import jax.numpy as jnp
from jax import lax
from jax.experimental import pallas as pl
from jax.experimental.pallas import tpu as pltpu
from jax.experimental.pallas import tpu_sc as plsc

S = 128
D = 512
B = 16384

NC = 2
NS = 16
L = 16
NW = NC * NS
BPW = B // NW
CH = 32
NCH = BPW // CH
NBUF = 4


def _body(days_hbm, w_hbm, out_hbm, idx_v, table_v, *rest):
    bufs = rest[:NBUF]
    gsems = rest[NBUF:2 * NBUF]
    hsems = rest[2 * NBUF:3 * NBUF]
    osems = rest[3 * NBUF:]
    wid = lax.axis_index("s") * NC + lax.axis_index("c")
    base = wid * BPW

    @pl.when(lax.axis_index("s") == 0)
    def _load_table():
        pltpu.sync_copy(w_hbm, table_v)

    pltpu.sync_copy(days_hbm.at[pl.ds(base, BPW)], idx_v)

    for i in range(BPW // L):
        d = idx_v[pl.ds(i * L, L)]
        hi = jnp.clip(d, 1, S - 1)
        lo = hi - 1
        alpha = jnp.clip(d - lo, 0, 1)
        idx_v[pl.ds(i * L, L)] = lo + alpha
    plsc.subcore_barrier()

    XH = 8

    def expand(c):
        b = c % NBUF
        buf = bufs[b]
        hcopy = pltpu.async_copy(
            w_hbm.at[idx_v.at[pl.ds(c * CH, XH)]],
            buf.at[pl.ds(0, XH)], hsems[b])
        for g in range(CH // L):
            rvec = idx_v[pl.ds(c * CH + g * L, L)]
            for l in range(L):
                if g * L + l < XH:
                    continue
                r = rvec[l]
                pltpu.async_copy(table_v.at[r], buf.at[g * L + l], gsems[b])
        drain = pltpu.make_async_copy(
            w_hbm.at[pl.ds(0, CH - XH)], buf.at[pl.ds(XH, CH - XH)], gsems[b])
        return (drain, hcopy)

    rows = [None] * NBUF
    outs = [None] * NBUF
    rows[0] = expand(0)
    for c in range(NCH):
        b = c % NBUF
        nb = (c + 1) % NBUF
        if c + 1 < NCH:
            if outs[nb] is not None:
                outs[nb].wait()
            rows[nb] = expand(c + 1)
        rows[b][0].wait()
        rows[b][1].wait()
        outs[b] = pltpu.async_copy(
            bufs[b], out_hbm.at[pl.ds(base + c * CH, CH)], osems[b]
        )
    for o in outs:
        if o is not None:
            o.wait()


@jax.jit
def _gather_rows(days, w):
    mesh = plsc.VectorSubcoreMesh(core_axis_name="c", subcore_axis_name="s")
    return pl.kernel(
        _body,
        out_type=jax.ShapeDtypeStruct((B, D), jnp.float32),
        mesh=mesh,
        scratch_types=[
            pltpu.VMEM((BPW,), jnp.int32),
            pltpu.VMEM_SHARED((S, D), jnp.float32),
            *[pltpu.VMEM((CH, D), jnp.float32) for _ in range(NBUF)],
            *[pltpu.SemaphoreType.DMA for _ in range(3 * NBUF)],
        ],
    )(days, w)


def kernel(days, W, session_days, sorted_order):
    return _gather_rows(days, W)


# speedup vs baseline: 1.1419x; 1.1419x over previous
import jax
import jax.numpy as jnp
from jax import lax
from jax.experimental import pallas as pl
from jax.experimental.pallas import tpu as pltpu
from jax.experimental.pallas import tpu_sc as plsc

S = 128
D = 512
B = 16384

NC = 2
NS = 16
L = 16
NW = NC * NS
BPW = B // NW
CH = 64
NCH = BPW // CH
NBUF = 3


def _body(days_hbm, w_hbm, out_hbm, idx_v, table_v, *rest):
    bufs = rest[:NBUF]
    gsems = rest[NBUF:2 * NBUF]
    osems = rest[2 * NBUF:]
    wid = lax.axis_index("s") * NC + lax.axis_index("c")
    base = wid * BPW

    @pl.when(lax.axis_index("s") == 0)
    def _load_table():
        pltpu.sync_copy(w_hbm, table_v)

    pltpu.sync_copy(days_hbm.at[pl.ds(base, BPW)], idx_v)

    for i in range(BPW // L):
        d = idx_v[pl.ds(i * L, L)]
        hi = jnp.clip(d, 1, S - 1)
        lo = hi - 1
        alpha = jnp.clip(d - lo, 0, 1)
        idx_v[pl.ds(i * L, L)] = lo + alpha
    plsc.subcore_barrier()

    def expand(c):
        b = c % NBUF
        buf = bufs[b]
        for g in range(CH // L):
            rvec = idx_v[pl.ds(c * CH + g * L, L)]
            for l in range(L):
                r = rvec[l]
                pltpu.async_copy(table_v.at[r], buf.at[g * L + l], gsems[b])
        return pltpu.make_async_copy(
            w_hbm.at[pl.ds(0, CH)], buf, gsems[b])

    rows = [None] * NBUF
    outs = [None] * NBUF
    rows[0] = expand(0)
    for c in range(NCH):
        b = c % NBUF
        nb = (c + 1) % NBUF
        if c + 1 < NCH:
            if outs[nb] is not None:
                outs[nb].wait()
            rows[nb] = expand(c + 1)
        rows[b].wait()
        outs[b] = pltpu.async_copy(
            bufs[b], out_hbm.at[pl.ds(base + c * CH, CH)], osems[b]
        )
    for o in outs:
        if o is not None:
            o.wait()


@jax.jit
def _gather_rows(days, w):
    mesh = plsc.VectorSubcoreMesh(core_axis_name="c", subcore_axis_name="s")
    return pl.kernel(
        _body,
        out_type=jax.ShapeDtypeStruct((B, D), jnp.float32),
        mesh=mesh,
        scratch_types=[
            pltpu.VMEM((BPW,), jnp.int32),
            pltpu.VMEM_SHARED((S, D), jnp.float32),
            *[pltpu.VMEM((CH, D), jnp.float32) for _ in range(NBUF)],
            *[pltpu.SemaphoreType.DMA for _ in range(2 * NBUF)],
        ],
    )(days, w)


def kernel(days, W, session_days, sorted_order):
    return _gather_rows(days, W)


# speedup vs baseline: 1.1845x; 1.0373x over previous
import jax
import jax.numpy as jnp
from jax import lax
from jax.experimental import pallas as pl
from jax.experimental.pallas import tpu as pltpu
from jax.experimental.pallas import tpu_sc as plsc

S = 128
D = 512
B = 16384

NC = 2
NS = 16
L = 16
NW = NC * NS
BPW = B // NW
CH = 32
NCH = BPW // CH
NBUF = 4


def _body(days_hbm, w_hbm, out_hbm, idx_v, table_v, *rest):
    tsem = rest[0]
    bufs = rest[1:1 + NBUF]
    gsems = rest[1 + NBUF:1 + 2 * NBUF]
    osems = rest[1 + 2 * NBUF:]
    wid = lax.axis_index("s") * NC + lax.axis_index("c")
    base = wid * BPW

    @pl.when(lax.axis_index("s") == 0)
    def _load_table():
        pltpu.async_copy(w_hbm, table_v, tsem)

    pltpu.sync_copy(days_hbm.at[pl.ds(base, BPW)], idx_v)

    for i in range(BPW // L):
        d = idx_v[pl.ds(i * L, L)]
        hi = jnp.clip(d, 1, S - 1)
        lo = hi - 1
        alpha = jnp.clip(d - lo, 0, 1)
        idx_v[pl.ds(i * L, L)] = lo + alpha

    @pl.when(lax.axis_index("s") == 0)
    def _wait_table():
        pltpu.make_async_copy(w_hbm, table_v, tsem).wait()

    plsc.subcore_barrier()

    def expand(c):
        b = c % NBUF
        buf = bufs[b]
        for g in range(CH // L):
            rvec = idx_v[pl.ds(c * CH + g * L, L)]
            for l in range(L):
                r = rvec[l]
                pltpu.async_copy(table_v.at[r], buf.at[g * L + l], gsems[b])
        return pltpu.make_async_copy(
            w_hbm.at[pl.ds(0, CH)], buf, gsems[b])

    rows = [None] * NBUF
    outs = [None] * NBUF
    rows[0] = expand(0)
    for c in range(NCH):
        b = c % NBUF
        nb = (c + 1) % NBUF
        if c + 1 < NCH:
            if outs[nb] is not None:
                outs[nb].wait()
            rows[nb] = expand(c + 1)
        rows[b].wait()
        outs[b] = pltpu.async_copy(
            bufs[b], out_hbm.at[pl.ds(base + c * CH, CH)], osems[b]
        )
    for o in outs:
        if o is not None:
            o.wait()


@jax.jit
def _gather_rows(days, w):
    mesh = plsc.VectorSubcoreMesh(core_axis_name="c", subcore_axis_name="s")
    return pl.kernel(
        _body,
        out_type=jax.ShapeDtypeStruct((B, D), jnp.float32),
        mesh=mesh,
        scratch_types=[
            pltpu.VMEM((BPW,), jnp.int32),
            pltpu.VMEM_SHARED((S, D), jnp.float32),
            pltpu.SemaphoreType.DMA,
            *[pltpu.VMEM((CH, D), jnp.float32) for _ in range(NBUF)],
            *[pltpu.SemaphoreType.DMA for _ in range(2 * NBUF)],
        ],
    )(days, w)


def kernel(days, W, session_days, sorted_order):
    return _gather_rows(days, W)


# speedup vs baseline: 1.1960x; 1.0097x over previous
import jax
import jax.numpy as jnp
from jax import lax
from jax.experimental import pallas as pl
from jax.experimental.pallas import tpu as pltpu
from jax.experimental.pallas import tpu_sc as plsc

S = 128
D = 512
B = 16384

NC = 2
NS = 16
L = 16
NW = NC * NS
BPW = B // NW
CH = 16
NCH = BPW // CH
NBUF = 8


def _body(days_hbm, w_hbm, out_hbm, idx_v, table_v, *rest):
    tsem = rest[0]
    bufs = rest[1:1 + NBUF]
    gsems = rest[1 + NBUF:1 + 2 * NBUF]
    osems = rest[1 + 2 * NBUF:]
    wid = lax.axis_index("s") * NC + lax.axis_index("c")
    base = wid * BPW

    @pl.when(lax.axis_index("s") == 0)
    def _load_table():
        pltpu.async_copy(w_hbm, table_v, tsem)

    pltpu.sync_copy(days_hbm.at[pl.ds(base, BPW)], idx_v)

    for i in range(BPW // L):
        d = idx_v[pl.ds(i * L, L)]
        hi = jnp.clip(d, 1, S - 1)
        lo = hi - 1
        alpha = jnp.clip(d - lo, 0, 1)
        idx_v[pl.ds(i * L, L)] = lo + alpha

    @pl.when(lax.axis_index("s") == 0)
    def _wait_table():
        pltpu.make_async_copy(w_hbm, table_v, tsem).wait()

    plsc.subcore_barrier()

    def expand(c):
        b = c % NBUF
        buf = bufs[b]
        for g in range(CH // L):
            rvec = idx_v[pl.ds(c * CH + g * L, L)]
            for l in range(L):
                r = rvec[l]
                pltpu.async_copy(table_v.at[r], buf.at[g * L + l], gsems[b])
        return pltpu.make_async_copy(
            w_hbm.at[pl.ds(0, CH)], buf, gsems[b])

    rows = [None] * NBUF
    outs = [None] * NBUF
    rows[0] = expand(0)
    for c in range(NCH):
        b = c % NBUF
        nb = (c + 1) % NBUF
        if c + 1 < NCH:
            if outs[nb] is not None:
                outs[nb].wait()
            rows[nb] = expand(c + 1)
        rows[b].wait()
        outs[b] = pltpu.async_copy(
            bufs[b], out_hbm.at[pl.ds(base + c * CH, CH)], osems[b]
        )
    for o in outs:
        if o is not None:
            o.wait()


@jax.jit
def _gather_rows(days, w):
    mesh = plsc.VectorSubcoreMesh(core_axis_name="c", subcore_axis_name="s")
    return pl.kernel(
        _body,
        out_type=jax.ShapeDtypeStruct((B, D), jnp.float32),
        mesh=mesh,
        scratch_types=[
            pltpu.VMEM((BPW,), jnp.int32),
            pltpu.VMEM_SHARED((S, D), jnp.float32),
            pltpu.SemaphoreType.DMA,
            *[pltpu.VMEM((CH, D), jnp.float32) for _ in range(NBUF)],
            *[pltpu.SemaphoreType.DMA for _ in range(2 * NBUF)],
        ],
    )(days, w)


def kernel(days, W, session_days, sorted_order):
    return _gather_rows(days, W)
